# Initial kernel scaffold; baseline (speedup 1.0000x reference)
import jax
import jax.numpy as jnp
from jax.experimental import pallas as pl


def kernel(x, Wb, bb, Wg, W1, b1, W2, b2, Wr1, br1, Wr2, br2, Wc, bc, Wh, bh):
    raise NotImplementedError("write your pallas kernel here")


# speedup vs baseline: 1.0780x; 1.0780x over previous
import functools

import jax
import jax.numpy as jnp
from jax.experimental import pallas as pl
from jax.experimental.pallas import tpu as pltpu

T = 2048
H = 1024
F = 4096
E = 8
K = 2
CAP = 640
NCLS = 10
SLOTS = E * CAP
PAD_SLOTS = SLOTS + 8
LANES = 128

TT = 256
NT = T // TT
FC = 512
NF = F // FC


def _ka_body(x_ref, Wb_ref, bb_ref, Wg_ref,
             h_ref, s0_ref, s1_ref, w0_ref, w1_ref, laux_ref,
             base_ref, me_ref, ce_ref):
    ti = pl.program_id(0)

    @pl.when(ti == 0)
    def _():
        base_ref[...] = jnp.zeros((1, LANES), jnp.float32)
        me_ref[...] = jnp.zeros((1, LANES), jnp.float32)
        ce_ref[...] = jnp.zeros((1, LANES), jnp.float32)

    xt = x_ref[...]
    ht = jnp.maximum(
        jnp.dot(xt, Wb_ref[...], preferred_element_type=jnp.float32)
        + bb_ref[...], 0.0)
    h_ref[...] = ht

    lanef = jax.lax.broadcasted_iota(
        jnp.int32, (TT, LANES), 1).astype(jnp.float32)
    elane = lanef < E
    lg = jnp.dot(ht, Wg_ref[...], preferred_element_type=jnp.float32)
    lg = jnp.where(elane, lg, -1e30)
    m = jnp.max(lg, axis=1, keepdims=True)
    el = jnp.where(elane, jnp.exp(lg - m), 0.0)
    probs = el / jnp.sum(el, axis=1, keepdims=True)

    p1 = jnp.max(probs, axis=1, keepdims=True)
    i1 = jnp.min(jnp.where(probs == p1, lanef, 1e9), axis=1, keepdims=True)
    oh0 = (lanef == i1).astype(jnp.float32)
    probs2 = jnp.where(lanef == i1, -1.0, probs)
    p2 = jnp.max(probs2, axis=1, keepdims=True)
    i2 = jnp.min(jnp.where(probs2 == p2, lanef, 1e9), axis=1, keepdims=True)
    oh1 = (lanef == i2).astype(jnp.float32)

    r = jax.lax.broadcasted_iota(jnp.int32, (TT, TT), 0)
    c = jax.lax.broadcasted_iota(jnp.int32, (TT, TT), 1)
    Lst = (r > c).astype(jnp.float32)
    ohsum = oh0 + oh1
    prevc = jnp.dot(Lst, ohsum, preferred_element_type=jnp.float32)
    basec = base_ref[...]
    loc0 = jnp.sum(oh0 * (prevc + basec), axis=1, keepdims=True)
    loc1 = jnp.sum(oh1 * (prevc + basec), axis=1, keepdims=True)

    base_ref[...] = basec + jnp.sum(ohsum, axis=0, keepdims=True)
    me_ref[...] += jnp.sum(probs, axis=0, keepdims=True)
    ce_ref[...] += jnp.sum(oh0, axis=0, keepdims=True)

    p12 = p1 + p2
    w0 = p1 / p12
    w1 = p2 / p12
    v0 = loc0 < CAP
    v1 = loc1 < CAP
    s0 = jnp.where(v0, i1 * CAP + loc0, float(SLOTS))
    s1 = jnp.where(v1, i2 * CAP + loc1, float(SLOTS))
    s0_ref[...] = s0.astype(jnp.int32)
    s1_ref[...] = s1.astype(jnp.int32)
    w0_ref[...] = jnp.where(v0, w0, 0.0)
    w1_ref[...] = jnp.where(v1, w1, 0.0)

    @pl.when(ti == NT - 1)
    def _():
        me = me_ref[...] / T
        ce = ce_ref[...] / T
        laux_ref[...] = E * jnp.sum(me * ce, axis=(0, 1), keepdims=True)


def _routing(x, Wb, bb, Wg_pad):
    return pl.pallas_call(
        _ka_body,
        grid=(NT,),
        in_specs=[
            pl.BlockSpec((TT, H), lambda t: (t, 0)),
            pl.BlockSpec((H, H), lambda t: (0, 0)),
            pl.BlockSpec((1, H), lambda t: (0, 0)),
            pl.BlockSpec((H, LANES), lambda t: (0, 0)),
        ],
        out_specs=[
            pl.BlockSpec((TT, H), lambda t: (t, 0)),
            pl.BlockSpec((TT, 1), lambda t: (t, 0)),
            pl.BlockSpec((TT, 1), lambda t: (t, 0)),
            pl.BlockSpec((TT, 1), lambda t: (t, 0)),
            pl.BlockSpec((TT, 1), lambda t: (t, 0)),
            pl.BlockSpec((1, 1), lambda t: (0, 0)),
        ],
        out_shape=[
            jax.ShapeDtypeStruct((T, H), jnp.float32),
            jax.ShapeDtypeStruct((T, 1), jnp.int32),
            jax.ShapeDtypeStruct((T, 1), jnp.int32),
            jax.ShapeDtypeStruct((T, 1), jnp.float32),
            jax.ShapeDtypeStruct((T, 1), jnp.float32),
            jax.ShapeDtypeStruct((1, 1), jnp.float32),
        ],
        scratch_shapes=[
            pltpu.VMEM((1, LANES), jnp.float32),
            pltpu.VMEM((1, LANES), jnp.float32),
            pltpu.VMEM((1, LANES), jnp.float32),
        ],
    )(x, Wb, bb, Wg_pad)


def _kb_body(s0_ref, s1_ref, h_ref, buf_ref):
    buf_ref[...] = jnp.zeros((PAD_SLOTS, H), jnp.float32)

    def body(t, carry):
        row = h_ref[pl.ds(t, 1), :]
        buf_ref[pl.ds(s0_ref[t], 1), :] = row
        buf_ref[pl.ds(s1_ref[t], 1), :] = row
        return carry

    jax.lax.fori_loop(0, T, body, 0)


def _dispatch(h, s0, s1):
    return pl.pallas_call(
        _kb_body,
        in_specs=[
            pl.BlockSpec(memory_space=pltpu.SMEM),
            pl.BlockSpec(memory_space=pltpu.SMEM),
            pl.BlockSpec((T, H), lambda: (0, 0)),
        ],
        out_specs=pl.BlockSpec((PAD_SLOTS, H), lambda: (0, 0)),
        out_shape=jax.ShapeDtypeStruct((PAD_SLOTS, H), jnp.float32),
        compiler_params=pltpu.CompilerParams(
            vmem_limit_bytes=100 * 1024 * 1024),
    )(s0, s1, h)


def _kc_body(buf_ref, W1_ref, b1_ref, W2_ref, b2_ref, eout_ref):
    nf = pl.program_id(1)
    hmid = jnp.maximum(
        jnp.dot(buf_ref[...], W1_ref[0], preferred_element_type=jnp.float32)
        + b1_ref[0], 0.0)
    contrib = jnp.dot(hmid, W2_ref[0], preferred_element_type=jnp.float32)

    @pl.when(nf == 0)
    def _():
        eout_ref[...] = contrib + b2_ref[0]

    @pl.when(nf > 0)
    def _():
        eout_ref[...] += contrib


def _expert_ffn(buf, W1, b1r, W2, b2r):
    return pl.pallas_call(
        _kc_body,
        grid=(E, NF),
        in_specs=[
            pl.BlockSpec((CAP, H), lambda e, nf: (e, 0)),
            pl.BlockSpec((1, H, FC), lambda e, nf: (e, 0, nf)),
            pl.BlockSpec((1, 1, FC), lambda e, nf: (e, 0, nf)),
            pl.BlockSpec((1, FC, H), lambda e, nf: (e, nf, 0)),
            pl.BlockSpec((1, 1, H), lambda e, nf: (e, 0, 0)),
        ],
        out_specs=pl.BlockSpec((CAP, H), lambda e, nf: (e, 0)),
        out_shape=jax.ShapeDtypeStruct((SLOTS, H), jnp.float32),
        compiler_params=pltpu.CompilerParams(
            dimension_semantics=("arbitrary", "arbitrary"),
            vmem_limit_bytes=100 * 1024 * 1024),
    )(buf, W1, b1r, W2, b2r)


def _kd_body(s0_ref, s1_ref, w0_ref, w1_ref, eout_ref, moe_ref):
    def body(t, carry):
        a0 = jnp.minimum(s0_ref[t], SLOTS - 1)
        a1 = jnp.minimum(s1_ref[t], SLOTS - 1)
        r0 = eout_ref[pl.ds(a0, 1), :]
        r1 = eout_ref[pl.ds(a1, 1), :]
        moe_ref[pl.ds(t, 1), :] = r0 * w0_ref[t] + r1 * w1_ref[t]
        return carry

    jax.lax.fori_loop(0, T, body, 0)


def _combine(eout, s0, s1, w0, w1):
    return pl.pallas_call(
        _kd_body,
        in_specs=[
            pl.BlockSpec(memory_space=pltpu.SMEM),
            pl.BlockSpec(memory_space=pltpu.SMEM),
            pl.BlockSpec(memory_space=pltpu.SMEM),
            pl.BlockSpec(memory_space=pltpu.SMEM),
            pl.BlockSpec((SLOTS, H), lambda: (0, 0)),
        ],
        out_specs=pl.BlockSpec((T, H), lambda: (0, 0)),
        out_shape=jax.ShapeDtypeStruct((T, H), jnp.float32),
        compiler_params=pltpu.CompilerParams(
            vmem_limit_bytes=100 * 1024 * 1024),
    )(s0, s1, w0, w1, eout)


def _ke_body(h_ref, Wr1_ref, br1_ref, Wr2_ref, br2_ref,
             moe_ref, Wc_ref, bc_ref, Wh_ref, bh_ref,
             out_ref, acc_ref):
    nf = pl.program_id(1)
    ht = h_ref[...]
    mid = jnp.maximum(
        jnp.dot(ht, Wr1_ref[...], preferred_element_type=jnp.float32)
        + br1_ref[...], 0.0)
    contrib = jnp.dot(mid, Wr2_ref[...], preferred_element_type=jnp.float32)

    @pl.when(nf == 0)
    def _():
        acc_ref[...] = contrib

    @pl.when(nf > 0)
    def _():
        acc_ref[...] += contrib

    @pl.when(nf == NF - 1)
    def _():
        res = acc_ref[...] + br2_ref[...]
        lanef = jax.lax.broadcasted_iota(
            jnp.int32, (TT, LANES), 1).astype(jnp.float32)
        clane = lanef < 2
        cl = jnp.dot(ht, Wc_ref[...], preferred_element_type=jnp.float32)
        cl = jnp.where(clane, cl + bc_ref[...], -1e30)
        cm = jnp.max(cl, axis=1, keepdims=True)
        ce = jnp.where(clane, jnp.exp(cl - cm), 0.0)
        cp = ce / jnp.sum(ce, axis=1, keepdims=True)
        coef0 = jnp.sum(jnp.where(lanef == 0, cp, 0.0), axis=1, keepdims=True)
        coef1 = jnp.sum(jnp.where(lanef == 1, cp, 0.0), axis=1, keepdims=True)
        comb = moe_ref[...] * coef0 + res * coef1
        out_ref[...] = (
            jnp.dot(comb, Wh_ref[...], preferred_element_type=jnp.float32)
            + bh_ref[...])


def _final(h, moe, Wr1, br1r, Wr2, br2r, Wc_pad, bc_pad, Wh_pad, bh_pad):
    return pl.pallas_call(
        _ke_body,
        grid=(NT, NF),
        in_specs=[
            pl.BlockSpec((TT, H), lambda t, nf: (t, 0)),
            pl.BlockSpec((H, FC), lambda t, nf: (0, nf)),
            pl.BlockSpec((1, FC), lambda t, nf: (0, nf)),
            pl.BlockSpec((FC, H), lambda t, nf: (nf, 0)),
            pl.BlockSpec((1, H), lambda t, nf: (0, 0)),
            pl.BlockSpec((TT, H), lambda t, nf: (t, 0)),
            pl.BlockSpec((H, LANES), lambda t, nf: (0, 0)),
            pl.BlockSpec((1, LANES), lambda t, nf: (0, 0)),
            pl.BlockSpec((H, LANES), lambda t, nf: (0, 0)),
            pl.BlockSpec((1, LANES), lambda t, nf: (0, 0)),
        ],
        out_specs=pl.BlockSpec((TT, LANES), lambda t, nf: (t, 0)),
        out_shape=jax.ShapeDtypeStruct((T, LANES), jnp.float32),
        scratch_shapes=[pltpu.VMEM((TT, H), jnp.float32)],
        compiler_params=pltpu.CompilerParams(
            dimension_semantics=("arbitrary", "arbitrary"),
            vmem_limit_bytes=100 * 1024 * 1024),
    )(h, Wr1, br1r, Wr2, br2r, moe, Wc_pad, bc_pad, Wh_pad, bh_pad)


def kernel(x, Wb, bb, Wg, W1, b1, W2, b2, Wr1, br1, Wr2, br2, Wc, bc, Wh, bh):
    bbr = bb.reshape(1, H)
    Wg_pad = jnp.pad(Wg, ((0, 0), (0, LANES - E)))
    h, s0, s1, w0, w1, laux = _routing(x, Wb, bbr, Wg_pad)

    s0f = s0.reshape(T)
    s1f = s1.reshape(T)
    buf = _dispatch(h, s0f, s1f)

    eout = _expert_ffn(buf[:SLOTS], W1, b1.reshape(E, 1, F), W2,
                       b2.reshape(E, 1, H))

    moe = _combine(eout, s0f, s1f, w0.reshape(T), w1.reshape(T))

    Wc_pad = jnp.pad(Wc, ((0, 0), (0, LANES - 2)))
    bc_pad = jnp.pad(bc, (0, LANES - 2)).reshape(1, LANES)
    Wh_pad = jnp.pad(Wh, ((0, 0), (0, LANES - NCLS)))
    bh_pad = jnp.pad(bh, (0, LANES - NCLS)).reshape(1, LANES)
    out_pad = _final(h, moe, Wr1, br1.reshape(1, F), Wr2,
                     br2.reshape(1, H), Wc_pad, bc_pad, Wh_pad, bh_pad)

    return out_pad[:, :NCLS], laux.reshape(())
